# baseline (speedup 1.0000x reference)
import functools

import jax
import jax.numpy as jnp
from jax import lax
from jax.experimental import pallas as pl
from jax.experimental.pallas import tpu as pltpu
from jax.experimental.pallas import tpu_sc as plsc

_ALPHA = 1.5
_DEPTH = 6
_NUM_LEAVES = 64
_TOP_K = 2
_N_ITER = 30
_LANES = 16


def _entmax_prep_body(fs_ref, sel_ref):
    x = fs_ref[...] * (_ALPHA - 1.0)
    max_val = jnp.max(x, axis=-1, keepdims=True)
    tau_lo = max_val - 1.0
    tau_hi = max_val

    def p_fn(tau):
        c = jnp.maximum(x - tau, 0.0)
        return c * c

    f_lo = jnp.sum(p_fn(tau_lo), axis=-1, keepdims=True) - 1.0
    for _ in range(_N_ITER):
        tau_m = 0.5 * (tau_lo + tau_hi)
        f_m = jnp.sum(p_fn(tau_m), axis=-1, keepdims=True) - 1.0
        same_sign = (f_m * f_lo) >= 0.0
        tau_lo = jnp.where(same_sign, tau_m, tau_lo)
        f_lo = jnp.where(same_sign, f_m, f_lo)
        tau_hi = jnp.where(same_sign, tau_hi, tau_m)
    p = p_fn(0.5 * (tau_lo + tau_hi))
    sel_ref[...] = p / jnp.sum(p, axis=-1, keepdims=True)


def _router_body(x_ref, selt_ref, thr_ref, lt_ref, w_ref_, b_ref, logits_ref):
    t = x_ref.shape[1]
    fc = jnp.dot(x_ref[0], selt_ref[...], preferred_element_type=jnp.float32)
    fct = fc.T
    temp = jnp.exp(lt_ref[...])
    scaled = (fct - thr_ref[...]) / temp
    u = scaled * (_ALPHA - 1.0)
    max_val = jnp.maximum(u, 0.0)
    tau_lo = max_val - 1.0
    tau_hi = max_val

    def p2_fn(tau):
        c0 = jnp.maximum(u - tau, 0.0)
        c1 = jnp.maximum(-tau, 0.0)
        return c0 * c0, c1 * c1

    p0, p1 = p2_fn(tau_lo)
    f_lo = (p0 + p1) - 1.0
    for _ in range(_N_ITER):
        tau_m = 0.5 * (tau_lo + tau_hi)
        p0, p1 = p2_fn(tau_m)
        f_m = (p0 + p1) - 1.0
        same_sign = (f_m * f_lo) >= 0.0
        tau_lo = jnp.where(same_sign, tau_m, tau_lo)
        f_lo = jnp.where(same_sign, f_m, f_lo)
        tau_hi = jnp.where(same_sign, tau_hi, tau_m)
    p0, p1 = p2_fn(0.5 * (tau_lo + tau_hi))
    right = p0 / (p0 + p1)
    left = 1.0 - right
    bits_i = jax.lax.broadcasted_iota(jnp.int32, (_NUM_LEAVES, 1), 0)
    leaf = jnp.ones((_NUM_LEAVES, t), dtype=jnp.float32)
    for i in range(_DEPTH):
        bit_col = ((bits_i >> (_DEPTH - 1 - i)) & 1) == 1
        r_i = right[i:i + 1, :]
        l_i = left[i:i + 1, :]
        leaf = leaf * jnp.where(bit_col, r_i, l_i)
    logits = jnp.dot(w_ref_[...], leaf, preferred_element_type=jnp.float32)
    logits_ref[...] = (logits + b_ref[...])[None]


def _sc_topk_body(logits_hbm, idx_hbm, w_hbm, buf, idxbuf, wbuf):
    wid = lax.axis_index("s") * 2 + lax.axis_index("c")
    l = logits_hbm.shape[2]
    chunk = (logits_hbm.shape[0] * l) // 32
    per_batch = l // chunk
    bi = wid // per_batch
    t0 = (wid % per_batch) * chunk
    pltpu.sync_copy(logits_hbm.at[bi, :, pl.ds(t0, chunk)], buf)

    def body(i, carry):
        base = i * _LANES
        m1 = buf[0, pl.ds(base, _LANES)]
        i1 = jnp.zeros((_LANES,), jnp.int32)
        m2 = jnp.full((_LANES,), -jnp.inf, jnp.float32)
        i2 = jnp.zeros((_LANES,), jnp.int32)
        for e in range(1, _NUM_LEAVES):
            v = buf[e, pl.ds(base, _LANES)]
            e_vec = jnp.full((_LANES,), e, jnp.int32)
            gt1 = v > m1
            gt2 = v > m2
            new_m2 = jnp.where(gt1, m1, jnp.where(gt2, v, m2))
            new_i2 = jnp.where(gt1, i1, jnp.where(gt2, e_vec, i2))
            m1 = jnp.where(gt1, v, m1)
            i1 = jnp.where(gt1, e_vec, i1)
            m2 = new_m2
            i2 = new_i2
        ex = jnp.exp(m2 - m1)
        z = 1.0 + ex
        idxbuf[0, pl.ds(base, _LANES)] = i1
        idxbuf[1, pl.ds(base, _LANES)] = i2
        wbuf[0, pl.ds(base, _LANES)] = 1.0 / z
        wbuf[1, pl.ds(base, _LANES)] = ex / z
        return carry

    lax.fori_loop(0, chunk // _LANES, body, 0)
    pltpu.sync_copy(idxbuf, idx_hbm.at[bi, :, pl.ds(t0, chunk)])
    pltpu.sync_copy(wbuf, w_hbm.at[bi, :, pl.ds(t0, chunk)])


@functools.partial(jax.jit, static_argnames=("block_t",))
def _run(hidden_states, feature_selectors, thresholds, log_temperatures,
         w_leaf, b_leaf, block_t=4096):
    b, l, h = hidden_states.shape
    num_experts = w_leaf.shape[0]

    sel = pl.pallas_call(
        _entmax_prep_body,
        out_shape=jax.ShapeDtypeStruct(feature_selectors.shape, jnp.float32),
    )(feature_selectors)

    selt = jnp.pad(sel.T, ((0, 0), (0, 8 - _DEPTH)))
    thr_col = jnp.pad(thresholds, ((0, 8 - _DEPTH), (0, 0)))
    lt_col = jnp.pad(log_temperatures, ((0, 8 - _DEPTH), (0, 0)))
    b_col = b_leaf.reshape(num_experts, 1)

    grid = (b, l // block_t)
    logits = pl.pallas_call(
        _router_body,
        grid=grid,
        in_specs=[
            pl.BlockSpec((1, block_t, h), lambda i, j: (i, j, 0)),
            pl.BlockSpec((h, 8), lambda i, j: (0, 0)),
            pl.BlockSpec((8, 1), lambda i, j: (0, 0)),
            pl.BlockSpec((8, 1), lambda i, j: (0, 0)),
            pl.BlockSpec((num_experts, _NUM_LEAVES), lambda i, j: (0, 0)),
            pl.BlockSpec((num_experts, 1), lambda i, j: (0, 0)),
        ],
        out_specs=pl.BlockSpec((1, num_experts, block_t),
                               lambda i, j: (i, 0, j)),
        out_shape=jax.ShapeDtypeStruct((b, num_experts, l), jnp.float32),
        compiler_params=pltpu.CompilerParams(
            dimension_semantics=("arbitrary", "arbitrary"),
        ),
    )(hidden_states, selt, thr_col, lt_col, w_leaf, b_col)

    chunk = (b * l) // 32
    sc_topk = functools.partial(
        pl.kernel,
        out_type=[
            jax.ShapeDtypeStruct((b, _TOP_K, l), jnp.int32),
            jax.ShapeDtypeStruct((b, _TOP_K, l), jnp.float32),
        ],
        scratch_types=[
            pltpu.VMEM((num_experts, chunk), jnp.float32),
            pltpu.VMEM((_TOP_K, chunk), jnp.int32),
            pltpu.VMEM((_TOP_K, chunk), jnp.float32),
        ],
        mesh=plsc.VectorSubcoreMesh(core_axis_name="c", subcore_axis_name="s"),
    )(_sc_topk_body)
    idx, w = sc_topk(logits)

    return (jnp.transpose(logits, (0, 2, 1)),
            jnp.transpose(idx, (0, 2, 1)),
            jnp.transpose(w, (0, 2, 1)))


def kernel(hidden_states, feature_selectors, thresholds, log_temperatures,
           w_leaf, b_leaf):
    return _run(hidden_states, feature_selectors, thresholds,
                log_temperatures, w_leaf, b_leaf)
